# baseline (speedup 1.0000x reference)
import functools

import jax
import jax.numpy as jnp
from jax import lax
from jax.experimental import pallas as pl
from jax.experimental.pallas import tpu as pltpu
from jax.experimental.pallas import tpu_sc as plsc

NCLASS = 1000
NFEAT = 128
BATCH = 16384
LANES = 16
NC = 2
NS = 16
NW = NC * NS
BPW = BATCH // NW
CHUNK = 128
NCHUNK = BPW // CHUNK


def _sc_body(feat_hbm, logits_hbm, y_hbm, centers_hbm, out_hbm,
             y_v, idx_v, picked_v, feat_v, cent_v, out_v, sem_p, sem_c):
    wid = lax.axis_index("s") * NC + lax.axis_index("c")
    base = wid * BPW

    pltpu.sync_copy(y_hbm.at[pl.ds(base * 1, BPW)], y_v)

    for j in range(BPW // LANES):
        y16 = y_v[pl.ds(j * LANES, LANES)]
        rows = (base + j * LANES) + lax.iota(jnp.int32, LANES)
        idx_v[pl.ds(j * LANES, LANES)] = rows * NCLASS + y16

    for c in range(NCHUNK):
        pltpu.async_copy(
            logits_hbm.at[idx_v.at[pl.ds(c * CHUNK, CHUNK)]],
            picked_v.at[pl.ds(c * CHUNK, CHUNK)],
            sem_p,
        )

    acc_sq = jnp.zeros((LANES,), jnp.float32)
    for c in range(NCHUNK):
        pltpu.sync_copy(
            feat_hbm.at[pl.ds((base + c * CHUNK) * NFEAT, CHUNK * NFEAT)],
            feat_v,
        )
        pltpu.async_copy(
            centers_hbm.at[y_v.at[pl.ds(c * CHUNK, CHUNK)]],
            cent_v,
            sem_c,
        ).wait()

        def row_body(r, acc):
            for j in range(NFEAT // LANES):
                d = (feat_v[pl.ds(r * NFEAT + j * LANES, LANES)]
                     - cent_v[r, pl.ds(j * LANES, LANES)])
                acc = acc + d * d
            return acc

        acc_sq = lax.fori_loop(0, CHUNK, row_body, acc_sq)

    for c in range(NCHUNK):
        pltpu.make_async_copy(
            logits_hbm.at[idx_v.at[pl.ds(c * CHUNK, CHUNK)]],
            picked_v.at[pl.ds(c * CHUNK, CHUNK)],
            sem_p,
        ).wait()
    acc_p = jnp.zeros((LANES,), jnp.float32)
    for j in range(BPW // LANES):
        acc_p = acc_p + picked_v[pl.ds(j * LANES, LANES)]

    out_v[...] = 0.25 * acc_sq - acc_p
    pltpu.sync_copy(out_v, out_hbm.at[wid])


@jax.jit
def _sc_loss(feat_flat, logits_flat, y_i32, centers):
    mesh = plsc.VectorSubcoreMesh(core_axis_name="c", subcore_axis_name="s")
    partials = pl.kernel(
        _sc_body,
        out_type=jax.ShapeDtypeStruct((NW, LANES), jnp.float32),
        mesh=mesh,
        scratch_types=[
            pltpu.VMEM((BPW,), jnp.int32),
            pltpu.VMEM((BPW,), jnp.int32),
            pltpu.VMEM((BPW,), jnp.float32),
            pltpu.VMEM((CHUNK * NFEAT,), jnp.float32),
            pltpu.VMEM((CHUNK, NFEAT), jnp.float32),
            pltpu.VMEM((LANES,), jnp.float32),
            pltpu.SemaphoreType.DMA,
            pltpu.SemaphoreType.DMA,
        ],
    )(feat_flat, logits_flat, y_i32, centers)
    return jnp.sum(partials) / BATCH


def kernel(feat, logits, y, centers):
    feat_flat = feat.reshape(-1)
    logits_flat = logits.reshape(-1)
    y_i32 = y.astype(jnp.int32)
    return _sc_loss(feat_flat, logits_flat, y_i32, centers)


# speedup vs baseline: 1.0396x; 1.0396x over previous
import functools

import jax
import jax.numpy as jnp
from jax import lax
from jax.experimental import pallas as pl
from jax.experimental.pallas import tpu as pltpu
from jax.experimental.pallas import tpu_sc as plsc

NCLASS = 1000
NFEAT = 128
BATCH = 16384
LANES = 16
NC = 2
NS = 16
NW = NC * NS
BPW = BATCH // NW
CHUNK = 128
NCHUNK = BPW // CHUNK


def _sc_body(feat_hbm, logits_hbm, y_hbm, centers_hbm, out_hbm,
             y_v, idx_v, picked_v, feat_bufs, cent_bufs, out_v,
             sem_p, sem_f, sem_c):
    wid = lax.axis_index("s") * NC + lax.axis_index("c")
    base = wid * BPW

    pltpu.sync_copy(y_hbm.at[pl.ds(base * 1, BPW)], y_v)

    for j in range(BPW // LANES):
        y16 = y_v[pl.ds(j * LANES, LANES)]
        rows = (base + j * LANES) + lax.iota(jnp.int32, LANES)
        idx_v[pl.ds(j * LANES, LANES)] = rows * NCLASS + y16

    for c in range(NCHUNK):
        pltpu.async_copy(
            logits_hbm.at[idx_v.at[pl.ds(c * CHUNK, CHUNK)]],
            picked_v.at[pl.ds(c * CHUNK, CHUNK)],
            sem_p,
        )

    def chunk_copies(c):
        b = c % 2
        feat_cp = pltpu.make_async_copy(
            feat_hbm.at[pl.ds(base + c * CHUNK, CHUNK)],
            feat_bufs[b],
            sem_f[b],
        )
        cent_cp = pltpu.make_async_copy(
            centers_hbm.at[y_v.at[pl.ds(c * CHUNK, CHUNK)]],
            cent_bufs[b],
            sem_c[b],
        )
        return feat_cp, cent_cp

    for cp in chunk_copies(0):
        cp.start()
    acc_sq = jnp.zeros((LANES,), jnp.float32)
    for c in range(NCHUNK):
        if c + 1 < NCHUNK:
            for cp in chunk_copies(c + 1):
                cp.start()
        for cp in chunk_copies(c):
            cp.wait()
        feat_v, cent_v = feat_bufs[c % 2], cent_bufs[c % 2]

        def row_body(r, acc):
            for j in range(NFEAT // LANES):
                d = (feat_v[r, pl.ds(j * LANES, LANES)]
                     - cent_v[r, pl.ds(j * LANES, LANES)])
                acc = acc + d * d
            return acc

        acc_sq = lax.fori_loop(0, CHUNK, row_body, acc_sq)

    for c in range(NCHUNK):
        pltpu.make_async_copy(
            logits_hbm.at[idx_v.at[pl.ds(c * CHUNK, CHUNK)]],
            picked_v.at[pl.ds(c * CHUNK, CHUNK)],
            sem_p,
        ).wait()
    acc_p = jnp.zeros((LANES,), jnp.float32)
    for j in range(BPW // LANES):
        acc_p = acc_p + picked_v[pl.ds(j * LANES, LANES)]

    out_v[...] = 0.25 * acc_sq - acc_p
    pltpu.sync_copy(out_v, out_hbm.at[wid])


@jax.jit
def _sc_loss(feat2d, logits_flat, y_i32, centers):
    mesh = plsc.VectorSubcoreMesh(core_axis_name="c", subcore_axis_name="s")
    partials = pl.kernel(
        _sc_body,
        out_type=jax.ShapeDtypeStruct((NW, LANES), jnp.float32),
        mesh=mesh,
        scratch_types=[
            pltpu.VMEM((BPW,), jnp.int32),
            pltpu.VMEM((BPW,), jnp.int32),
            pltpu.VMEM((BPW,), jnp.float32),
            [pltpu.VMEM((CHUNK, NFEAT), jnp.float32)] * 2,
            [pltpu.VMEM((CHUNK, NFEAT), jnp.float32)] * 2,
            pltpu.VMEM((LANES,), jnp.float32),
            pltpu.SemaphoreType.DMA,
            [pltpu.SemaphoreType.DMA] * 2,
            [pltpu.SemaphoreType.DMA] * 2,
        ],
    )(feat2d, logits_flat, y_i32, centers)
    return jnp.sum(partials) / BATCH


def kernel(feat, logits, y, centers):
    logits_flat = logits.reshape(-1)
    y_i32 = y.astype(jnp.int32)
    return _sc_loss(feat, logits_flat, y_i32, centers)


# speedup vs baseline: 1.4127x; 1.3589x over previous
import functools

import jax
import jax.numpy as jnp
from jax import lax
from jax.experimental import pallas as pl
from jax.experimental.pallas import tpu as pltpu
from jax.experimental.pallas import tpu_sc as plsc

NCLASS = 1000
NFEAT = 128
BATCH = 16384
LANES = 16
NC = 2
NS = 16
NW = NC * NS
BPW = BATCH // NW
CHUNK = 128
NCHUNK = BPW // CHUNK

TC_BLOCK = 512
TC_GRID = BATCH // TC_BLOCK


def _sc_body(feat_hbm, y_hbm, centers_hbm, out_hbm,
             y_v, feat_bufs, cent_bufs, out_v, sem_f, sem_c):
    wid = lax.axis_index("s") * NC + lax.axis_index("c")
    base = wid * BPW

    pltpu.sync_copy(y_hbm.at[pl.ds(base, BPW)], y_v)

    def chunk_copies(c):
        b = c % 2
        feat_cp = pltpu.make_async_copy(
            feat_hbm.at[pl.ds(base + c * CHUNK, CHUNK)],
            feat_bufs[b],
            sem_f[b],
        )
        cent_cp = pltpu.make_async_copy(
            centers_hbm.at[y_v.at[pl.ds(c * CHUNK, CHUNK)]],
            cent_bufs[b],
            sem_c[b],
        )
        return feat_cp, cent_cp

    for cp in chunk_copies(0):
        cp.start()
    acc_sq = jnp.zeros((LANES,), jnp.float32)
    for c in range(NCHUNK):
        if c + 1 < NCHUNK:
            for cp in chunk_copies(c + 1):
                cp.start()
        for cp in chunk_copies(c):
            cp.wait()
        feat_v, cent_v = feat_bufs[c % 2], cent_bufs[c % 2]

        def row_body(r, acc):
            for j in range(NFEAT // LANES):
                d = (feat_v[r, pl.ds(j * LANES, LANES)]
                     - cent_v[r, pl.ds(j * LANES, LANES)])
                acc = acc + d * d
            return acc

        acc_sq = lax.fori_loop(0, CHUNK, row_body, acc_sq)

    out_v[...] = acc_sq
    pltpu.sync_copy(out_v, out_hbm.at[wid])


def _tc_body(logits_ref, y_ref, out_ref):
    labels = y_ref[0, 0, :]
    cols = lax.broadcasted_iota(jnp.int32, (TC_BLOCK, NCLASS), 1)
    mask = cols == labels[:, None]
    picked = jnp.where(mask, logits_ref[...], 0.0)
    out_ref[...] = jnp.sum(picked, axis=0)[None, None, :]


@jax.jit
def _loss(feat2d, logits2d, y_i32, centers):
    mesh = plsc.VectorSubcoreMesh(core_axis_name="c", subcore_axis_name="s")
    sq_partials = pl.kernel(
        _sc_body,
        out_type=jax.ShapeDtypeStruct((NW, LANES), jnp.float32),
        mesh=mesh,
        scratch_types=[
            pltpu.VMEM((BPW,), jnp.int32),
            [pltpu.VMEM((CHUNK, NFEAT), jnp.float32)] * 2,
            [pltpu.VMEM((CHUNK, NFEAT), jnp.float32)] * 2,
            pltpu.VMEM((LANES,), jnp.float32),
            [pltpu.SemaphoreType.DMA] * 2,
            [pltpu.SemaphoreType.DMA] * 2,
        ],
    )(feat2d, y_i32, centers)

    y3 = y_i32.reshape(TC_GRID, 1, TC_BLOCK)
    picked_partials = pl.pallas_call(
        _tc_body,
        grid=(TC_GRID,),
        in_specs=[
            pl.BlockSpec((TC_BLOCK, NCLASS), lambda i: (i, 0)),
            pl.BlockSpec((1, 1, TC_BLOCK), lambda i: (i, 0, 0)),
        ],
        out_specs=pl.BlockSpec((1, 1, NCLASS), lambda i: (i, 0, 0)),
        out_shape=jax.ShapeDtypeStruct((TC_GRID, 1, NCLASS), jnp.float32),
    )(logits2d, y3)

    return (0.25 * jnp.sum(sq_partials) - jnp.sum(picked_partials)) / BATCH


def kernel(feat, logits, y, centers):
    return _loss(feat, logits, y.astype(jnp.int32), centers)
